# Initial kernel scaffold; baseline (speedup 1.0000x reference)
import jax
import jax.numpy as jnp
from jax.experimental import pallas as pl


def kernel(nodes, pos, senders, receivers, edge_attr, glb, We1, be1, We2, be2, Wn1, bn1, Wn2, bn2, Wp1, bp1, Wpl):
    raise NotImplementedError("write your pallas kernel here")


# speedup vs baseline: 2.8801x; 2.8801x over previous
import dataclasses
import functools

import jax
import jax.numpy as jnp
from jax import lax
from jax.experimental import pallas as pl
from jax.experimental.pallas import tpu as pltpu
from jax.experimental.pallas import tpu_sc as plsc

N = 10000
E = 320000
D = 128
H = 128
DE = 16
DG = 16

NC = 2
NS = 16
NW = NC * NS
EPW = E // NW
W = 80
NCHUNK = EPW // W
NPAD = 10240
RPT = NPAD // NS

BN = 1000
BE = 512

_f32 = jnp.float32


def _silu(x):
    return x * jax.nn.sigmoid(x)


def _sc_compiler_params():
    cp = pltpu.CompilerParams()
    fields = pltpu.CompilerParams.__dataclass_fields__
    if "needs_layout_passes" in fields:
        cp = dataclasses.replace(cp, needs_layout_passes=False)
    if "use_tc_tiling_on_sc" in fields:
        cp = dataclasses.replace(cp, use_tc_tiling_on_sc=False)
    return cp


def _tca_body(nodes_ref, we1a_ref, we1b_ref, glb_ref, we1g_ref, be1_ref,
              a_ref, b_ref):
    c0 = jnp.dot(glb_ref[...], we1g_ref[...],
                 preferred_element_type=_f32) + be1_ref[...]
    x = nodes_ref[...]
    a_ref[...] = jnp.dot(x, we1a_ref[...], preferred_element_type=_f32) + c0
    b_ref[...] = jnp.dot(x, we1b_ref[...], preferred_element_type=_f32)


def _tca(nodes, we1a, we1b, glb, we1g, be1):
    full = lambda s: pl.BlockSpec(s, lambda i: (0, 0))
    return pl.pallas_call(
        _tca_body,
        grid=(N // BN,),
        in_specs=[
            pl.BlockSpec((BN, D), lambda i: (i, 0)),
            full((D, H)), full((D, H)), full((1, DG)), full((DG, H)),
            full((1, H)),
        ],
        out_specs=[
            pl.BlockSpec((BN, H), lambda i: (i, 0)),
            pl.BlockSpec((BN, H), lambda i: (i, 0)),
        ],
        out_shape=[
            jax.ShapeDtypeStruct((N, H), _f32),
            jax.ShapeDtypeStruct((N, H), _f32),
        ],
    )(nodes, we1a, we1b, glb, we1g, be1)


def _sc_gather_body(a_hbm, b_hbm, snd_hbm, rcv_hbm, px_hbm, py_hbm, pz_hbm,
                    pre_hbm, cd_hbm,
                    buf_a, buf_b, sidx, ridx, cd_buf, posx, posy, posz):
    cid = lax.axis_index("c")
    sid = lax.axis_index("s")
    base = (sid * NC + cid) * EPW

    @pl.loop(0, W)
    def _(e):
        cd_buf[e, pl.ds(0, 16)] = jnp.zeros((16,), _f32)

    pltpu.sync_copy(px_hbm, posx)
    pltpu.sync_copy(py_hbm, posy)
    pltpu.sync_copy(pz_hbm, posz)

    @pl.loop(0, NCHUNK)
    def _(i):
        off = base + i * W
        pltpu.sync_copy(snd_hbm.at[pl.ds(off, W)], sidx)
        pltpu.sync_copy(rcv_hbm.at[pl.ds(off, W)], ridx)
        pltpu.sync_copy(a_hbm.at[sidx], buf_a)
        pltpu.sync_copy(b_hbm.at[ridx], buf_b)

        @pl.loop(0, W)
        def _(e):
            for j in range(H // 16):
                sl = pl.ds(j * 16, 16)
                plsc.addupdate(buf_a.at[e, sl], buf_b[e, sl])

        @pl.loop(0, W // 16)
        def _(g):
            sl = pl.ds(g * 16, 16)
            sv = sidx[sl]
            rv = ridx[sl]
            eidx = g * 16 + lax.iota(jnp.int32, 16)
            dx = plsc.load_gather(posx, [sv]) - plsc.load_gather(posx, [rv])
            dy = plsc.load_gather(posy, [sv]) - plsc.load_gather(posy, [rv])
            dz = plsc.load_gather(posz, [sv]) - plsc.load_gather(posz, [rv])
            rad = dx * dx + dy * dy + dz * dz
            plsc.store_scatter(cd_buf, [eidx, jnp.full((16,), 0, jnp.int32)],
                               dx)
            plsc.store_scatter(cd_buf, [eidx, jnp.full((16,), 1, jnp.int32)],
                               dy)
            plsc.store_scatter(cd_buf, [eidx, jnp.full((16,), 2, jnp.int32)],
                               dz)
            plsc.store_scatter(cd_buf, [eidx, jnp.full((16,), 3, jnp.int32)],
                               rad)

        pltpu.sync_copy(buf_a, pre_hbm.at[pl.ds(off, W)])
        pltpu.sync_copy(cd_buf, cd_hbm.at[pl.ds(off, W)])


def _sc_gather(a, b, senders, receivers, px, py, pz):
    mesh = plsc.VectorSubcoreMesh(core_axis_name="c", subcore_axis_name="s")
    kern = pl.kernel(
        _sc_gather_body,
        out_type=[
            jax.ShapeDtypeStruct((E, H), _f32),
            jax.ShapeDtypeStruct((E, 16), _f32),
        ],
        mesh=mesh,
        scratch_types=[
            pltpu.VMEM((W, H), _f32),
            pltpu.VMEM((W, H), _f32),
            pltpu.VMEM((W,), jnp.int32),
            pltpu.VMEM((W,), jnp.int32),
            pltpu.VMEM((W, 16), _f32),
            pltpu.VMEM((N,), _f32),
            pltpu.VMEM((N,), _f32),
            pltpu.VMEM((N,), _f32),
        ],
        compiler_params=_sc_compiler_params(),
    )
    return kern(a, b, senders, receivers, px, py, pz)


def _tc2_body(pre_ref, cd_ref, ea_ref, wr_ref, we1e_ref, we2_ref, be2_ref,
              wp1_ref, bp1_ref, wpl_ref, msg_ref, trans_ref):
    rad = cd_ref[...][:, 3:4]
    pre1 = (pre_ref[...] + rad * wr_ref[...]
            + jnp.dot(ea_ref[...], we1e_ref[...], preferred_element_type=_f32))
    h = _silu(pre1)
    m = _silu(jnp.dot(h, we2_ref[...], preferred_element_type=_f32)
              + be2_ref[...])
    msg_ref[...] = m
    t = _silu(jnp.dot(m, wp1_ref[...], preferred_element_type=_f32)
              + bp1_ref[...])
    trans_ref[...] = jnp.dot(t, wpl_ref[...], preferred_element_type=_f32)


def _tc2(pre, cd16, edge_attr, wr, we1e, we2, be2, wp1, bp1, wpl):
    full = lambda s: pl.BlockSpec(s, lambda i: (0, 0))
    return pl.pallas_call(
        _tc2_body,
        grid=(E // BE,),
        in_specs=[
            pl.BlockSpec((BE, H), lambda i: (i, 0)),
            pl.BlockSpec((BE, 16), lambda i: (i, 0)),
            pl.BlockSpec((BE, DE), lambda i: (i, 0)),
            full((1, H)), full((DE, H)), full((H, H)), full((1, H)),
            full((H, H)), full((1, H)), full((H, 1)),
        ],
        out_specs=[
            pl.BlockSpec((BE, H), lambda i: (i, 0)),
            pl.BlockSpec((BE, 1), lambda i: (i, 0)),
        ],
        out_shape=[
            jax.ShapeDtypeStruct((E, H), _f32),
            jax.ShapeDtypeStruct((E, 1), _f32),
        ],
    )(pre, cd16, edge_attr, wr, we1e, we2, be2, wp1, bp1, wpl)


def _sc_scatter_body(msg_hbm, trans_hbm, snd_hbm, rcv_hbm, cd_hbm,
                     aggp_hbm, posp_hbm,
                     msg_buf, cd_buf, trans_buf, sidx, ridx,
                     acc_msg, acc_pos):
    cid = lax.axis_index("c")
    sid = lax.axis_index("s")
    base = (sid * NC + cid) * EPW
    zrow = sid * RPT

    @pl.loop(0, W)
    def _(e):
        for j in range(H // 16):
            msg_buf[e, pl.ds(j * 16, 16)] = jnp.zeros((16,), _f32)
        cd_buf[e, pl.ds(0, 16)] = jnp.zeros((16,), _f32)

    for k in range(RPT // W):
        pltpu.sync_copy(msg_buf, acc_msg.at[pl.ds(zrow + k * W, W)])
        pltpu.sync_copy(cd_buf, acc_pos.at[pl.ds(zrow + k * W, W)])

    plsc.subcore_barrier()

    @pl.loop(0, NCHUNK)
    def _(i):
        off = base + i * W
        pltpu.sync_copy(msg_hbm.at[pl.ds(off, W)], msg_buf)
        pltpu.sync_copy(trans_hbm.at[pl.ds(off, W)], trans_buf)
        pltpu.sync_copy(snd_hbm.at[pl.ds(off, W)], sidx)
        pltpu.sync_copy(rcv_hbm.at[pl.ds(off, W)], ridx)
        pltpu.sync_copy(cd_hbm.at[pl.ds(off, W)], cd_buf)

        @pl.loop(0, W // 16)
        def _(g):
            sl = pl.ds(g * 16, 16)
            tv = trans_buf[sl]
            eidx = g * 16 + lax.iota(jnp.int32, 16)
            for k in range(3):
                kvec = jnp.full((16,), k, jnp.int32)
                d = plsc.load_gather(cd_buf, [eidx, kvec])
                v = jnp.clip(d * tv, -100.0, 100.0)
                plsc.store_scatter(cd_buf, [eidx, kvec], v)

        pltpu.sync_copy(msg_buf, acc_msg.at[ridx], add=True)
        pltpu.sync_copy(cd_buf, acc_pos.at[sidx], add=True)

    plsc.subcore_barrier()

    pltpu.sync_copy(acc_msg.at[pl.ds(zrow, RPT)],
                    aggp_hbm.at[cid, pl.ds(zrow, RPT)])
    pltpu.sync_copy(acc_pos.at[pl.ds(zrow, RPT)],
                    posp_hbm.at[cid, pl.ds(zrow, RPT)])


def _sc_scatter(msg, trans1d, senders, receivers, cd16):
    mesh = plsc.VectorSubcoreMesh(core_axis_name="c", subcore_axis_name="s")
    kern = pl.kernel(
        _sc_scatter_body,
        out_type=[
            jax.ShapeDtypeStruct((NC, NPAD, H), _f32),
            jax.ShapeDtypeStruct((NC, NPAD, 16), _f32),
        ],
        mesh=mesh,
        scratch_types=[
            pltpu.VMEM((W, H), _f32),
            pltpu.VMEM((W, 16), _f32),
            pltpu.VMEM((W,), _f32),
            pltpu.VMEM((W,), jnp.int32),
            pltpu.VMEM((W,), jnp.int32),
            pltpu.VMEM_SHARED((NPAD, H), _f32),
            pltpu.VMEM_SHARED((NPAD, 16), _f32),
        ],
        compiler_params=_sc_compiler_params(),
    )
    return kern(msg, trans1d, senders, receivers, cd16)


def _tc4_body(nodes_ref, agg0_ref, agg1_ref, pos16_ref, pp0_ref, pp1_ref,
              wn1a_ref, wn1b_ref, bn1_ref, wn2_ref, bn2_ref,
              out_nodes_ref, out_pos_ref):
    x = nodes_ref[...]
    agg = agg0_ref[...] + agg1_ref[...]
    h = _silu(jnp.dot(x, wn1a_ref[...], preferred_element_type=_f32)
              + jnp.dot(agg, wn1b_ref[...], preferred_element_type=_f32)
              + bn1_ref[...])
    out_nodes_ref[...] = x + jnp.dot(h, wn2_ref[...],
                                     preferred_element_type=_f32) + bn2_ref[...]
    out_pos_ref[...] = pos16_ref[...] + pp0_ref[...] + pp1_ref[...]


def _tc4(nodes, agg0, agg1, pos16, pp0, pp1, wn1a, wn1b, bn1, wn2, bn2):
    full = lambda s: pl.BlockSpec(s, lambda i: (0, 0))
    return pl.pallas_call(
        _tc4_body,
        grid=(N // BN,),
        in_specs=[
            pl.BlockSpec((BN, D), lambda i: (i, 0)),
            pl.BlockSpec((BN, H), lambda i: (i, 0)),
            pl.BlockSpec((BN, H), lambda i: (i, 0)),
            pl.BlockSpec((BN, 16), lambda i: (i, 0)),
            pl.BlockSpec((BN, 16), lambda i: (i, 0)),
            pl.BlockSpec((BN, 16), lambda i: (i, 0)),
            full((D, H)), full((H, H)), full((1, H)), full((H, H)),
            full((1, H)),
        ],
        out_specs=[
            pl.BlockSpec((BN, H), lambda i: (i, 0)),
            pl.BlockSpec((BN, 16), lambda i: (i, 0)),
        ],
        out_shape=[
            jax.ShapeDtypeStruct((N, H), _f32),
            jax.ShapeDtypeStruct((N, 16), _f32),
        ],
    )(nodes, agg0, agg1, pos16, pp0, pp1, wn1a, wn1b, bn1, wn2, bn2)


def kernel(nodes, pos, senders, receivers, edge_attr, glb,
           We1, be1, We2, be2, Wn1, bn1, Wn2, bn2, Wp1, bp1, Wpl):
    we1a = We1[:D]
    we1b = We1[D:2 * D]
    wr = We1[2 * D:2 * D + 1]
    we1g = We1[2 * D + 1:2 * D + 1 + DG]
    we1e = We1[2 * D + 1 + DG:]

    be1r = be1.reshape(1, H)
    be2r = be2.reshape(1, H)
    bn1r = bn1.reshape(1, H)
    bn2r = bn2.reshape(1, H)
    bp1r = bp1.reshape(1, H)

    px = pos[:, 0]
    py = pos[:, 1]
    pz = pos[:, 2]
    pos16 = jnp.pad(pos, ((0, 0), (0, 13)))

    a, b = _tca(nodes, we1a, we1b, glb, we1g, be1r)
    pre, cd16 = _sc_gather(a, b, senders, receivers, px, py, pz)
    msg, trans = _tc2(pre, cd16, edge_attr,
                      wr, we1e, We2, be2r, Wp1, bp1r, Wpl)
    aggp, posp = _sc_scatter(msg, trans.reshape(E), senders, receivers, cd16)
    new_nodes, new_pos16 = _tc4(nodes, aggp[0], aggp[1], pos16,
                                posp[0], posp[1],
                                Wn1[:D], Wn1[D:], bn1r, Wn2, bn2r)
    return (new_nodes, msg, new_pos16[:, :3])
